# baseline (speedup 1.0000x reference)
import functools

import jax
import jax.numpy as jnp
from jax import lax
from jax.experimental import pallas as pl
from jax.experimental.pallas import tpu as pltpu
from jax.experimental.pallas import tpu_sc as plsc

_D = 1024
_WDIM = 20
_WPAD = 32
_VOCAB = 8
_BS = 2048
_L = 16
_R_TC = 28672


def _sc_stage(rows):
    info = plsc.get_sparse_core_info()
    nw = info.num_cores * info.num_subcores
    w_per = _R_TC // nw
    f_per = (rows - _R_TC) // nw
    wchunk = w_per // 2
    fchunk = 32
    mesh = plsc.VectorSubcoreMesh(core_axis_name="c", subcore_axis_name="s")

    @functools.partial(
        pl.kernel,
        mesh=mesh,
        compiler_params=pltpu.CompilerParams(needs_layout_passes=False),
        out_type=(
            jax.ShapeDtypeStruct((_R_TC, _WPAD), jnp.float32),
            jax.ShapeDtypeStruct((rows, _D + _WDIM), jnp.float32),
        ),
        scratch_types=[
            pltpu.VMEM((w_per,), jnp.int32),
            pltpu.VMEM((w_per,), jnp.int32),
            pltpu.VMEM((f_per,), jnp.int32),
            pltpu.VMEM((f_per,), jnp.int32),
            pltpu.VMEM((_VOCAB * _WPAD,), jnp.float32),
            pltpu.VMEM((wchunk, _WPAD), jnp.float32),
            pltpu.VMEM((fchunk, _D + _WDIM), jnp.float32),
        ],
    )
    def k(starts_hbm, ends_hbm, table_hbm, emb_hbm, wemb_hbm, outm_hbm,
          sv, ev, s2, e2, table_v, rows_v, buf):
        wid = lax.axis_index("s") * info.num_cores + lax.axis_index("c")
        lane = lax.iota(jnp.int32, _L)
        pltpu.sync_copy(table_hbm, table_v)

        wbase = wid * w_per
        pltpu.sync_copy(starts_hbm.at[pl.ds(wbase, w_per)], sv)
        pltpu.sync_copy(ends_hbm.at[pl.ds(wbase, w_per)], ev)

        for h in range(w_per // wchunk):
            def wbody(g, _):
                sl = pl.ds(h * wchunk + g * _L, _L)
                w_vec = ev[sl] - sv[sl]
                tbase = w_vec * _WPAD
                r_vec = g * _L + lane
                for c in range(_WDIM):
                    val = plsc.load_gather(table_v, [tbase + c])
                    plsc.store_scatter(
                        rows_v, [r_vec, jnp.full((_L,), c, jnp.int32)], val)
                return 0

            lax.fori_loop(0, wchunk // _L, wbody, 0, unroll=1)
            pltpu.sync_copy(
                rows_v, wemb_hbm.at[pl.ds(wbase + h * wchunk, wchunk)])

        fbase = _R_TC + wid * f_per
        pltpu.sync_copy(starts_hbm.at[pl.ds(fbase, f_per)], s2)
        pltpu.sync_copy(ends_hbm.at[pl.ds(fbase, f_per)], e2)

        for h in range(f_per // fchunk):
            row0 = fbase + h * fchunk
            pltpu.sync_copy(emb_hbm.at[pl.ds(row0, fchunk)],
                            buf.at[:, pl.ds(0, _D)])

            def fbody(g, _):
                sl = pl.ds(h * fchunk + g * _L, _L)
                w_vec = e2[sl] - s2[sl]
                tbase = w_vec * _WPAD
                r_vec = g * _L + lane
                for c in range(_WDIM):
                    val = plsc.load_gather(table_v, [tbase + c])
                    plsc.store_scatter(
                        buf, [r_vec, jnp.full((_L,), _D + c, jnp.int32)], val)
                return 0

            lax.fori_loop(0, fchunk // _L, fbody, 0, unroll=1)
            pltpu.sync_copy(buf, outm_hbm.at[pl.ds(row0, fchunk)])

    return k


def _tc_body(alias_ref, emb_ref, w_ref, out_ref):
    del alias_ref
    out_ref[:, :_D] = emb_ref[...]
    out_ref[:, _D:] = w_ref[:, :_WDIM]


def kernel(spans, span_embeddings, width_table):
    B, S, D = span_embeddings.shape
    rows = B * S
    starts = spans[..., 0].astype(jnp.int32).reshape(rows)
    ends = spans[..., 1].astype(jnp.int32).reshape(rows)
    table_pad = jnp.zeros((_VOCAB, _WPAD), jnp.float32).at[:, :_WDIM].set(width_table)
    emb = span_embeddings.reshape(rows, D)

    wemb, outm = _sc_stage(rows)(starts, ends, table_pad.reshape(-1), emb)

    out = pl.pallas_call(
        _tc_body,
        grid=(_R_TC // _BS,),
        in_specs=[
            pl.BlockSpec(memory_space=pltpu.MemorySpace.HBM),
            pl.BlockSpec((_BS, D), lambda i: (i, 0)),
            pl.BlockSpec((_BS, _WPAD), lambda i: (i, 0)),
        ],
        out_specs=pl.BlockSpec((_BS, D + _WDIM), lambda i: (i, 0)),
        out_shape=jax.ShapeDtypeStruct((rows, D + _WDIM), jnp.float32),
        input_output_aliases={0: 0},
    )(outm, emb, wemb)
    return out.reshape(B, S, D + _WDIM)


# speedup vs baseline: 1.0166x; 1.0166x over previous
import functools

import jax
import jax.numpy as jnp
from jax import lax
from jax.experimental import pallas as pl
from jax.experimental.pallas import tpu as pltpu
from jax.experimental.pallas import tpu_sc as plsc

_D = 1024
_WDIM = 20
_WPAD = 32
_VOCAB = 8
_BS = 2048
_L = 16


def _sc_gather(rows):
    info = plsc.get_sparse_core_info()
    nw = info.num_cores * info.num_subcores
    b_per_w = rows // nw
    mesh = plsc.VectorSubcoreMesh(core_axis_name="c", subcore_axis_name="s")

    chunk = 512

    @functools.partial(
        pl.kernel,
        mesh=mesh,
        compiler_params=pltpu.CompilerParams(needs_layout_passes=False),
        out_type=jax.ShapeDtypeStruct((rows, _WPAD), jnp.float32),
        scratch_types=[
            pltpu.VMEM((b_per_w,), jnp.int32),
            pltpu.VMEM((b_per_w,), jnp.int32),
            pltpu.VMEM((_VOCAB * _WPAD,), jnp.float32),
            pltpu.VMEM((chunk, _WPAD), jnp.float32),
        ],
    )
    def k(starts_hbm, ends_hbm, table_hbm, wemb_hbm,
          starts_v, ends_v, table_v, rows_v):
        wid = lax.axis_index("s") * info.num_cores + lax.axis_index("c")
        base = wid * b_per_w
        pltpu.sync_copy(starts_hbm.at[pl.ds(base, b_per_w)], starts_v)
        pltpu.sync_copy(ends_hbm.at[pl.ds(base, b_per_w)], ends_v)
        pltpu.sync_copy(table_hbm, table_v)
        lane = lax.iota(jnp.int32, _L)

        for h in range(b_per_w // chunk):
            def body(g, _):
                sl = pl.ds(h * chunk + g * _L, _L)
                w_vec = ends_v[sl] - starts_v[sl]
                tbase = w_vec * _WPAD
                r_vec = g * _L + lane
                for c in range(_WDIM):
                    val = plsc.load_gather(table_v, [tbase + c])
                    plsc.store_scatter(rows_v, [r_vec, jnp.full((_L,), c, jnp.int32)], val)
                return 0

            lax.fori_loop(0, chunk // _L, body, 0, unroll=1)
            pltpu.sync_copy(
                rows_v, wemb_hbm.at[pl.ds(base + h * chunk, chunk)])

    return k


def _tc_body(emb_ref, w_ref, out_ref):
    out_ref[:, :_D] = emb_ref[...]
    out_ref[:, _D:] = w_ref[:, :_WDIM]


def kernel(spans, span_embeddings, width_table):
    B, S, D = span_embeddings.shape
    rows = B * S
    nb = rows // _BS
    starts = spans[..., 0].astype(jnp.int32).reshape(rows)
    ends = spans[..., 1].astype(jnp.int32).reshape(rows)
    table_pad = jnp.zeros((_VOCAB, _WPAD), jnp.float32).at[:, :_WDIM].set(width_table)
    emb = span_embeddings.reshape(rows, D)

    wemb = _sc_gather(rows)(starts, ends, table_pad.reshape(-1))

    out = pl.pallas_call(
        _tc_body,
        grid=(nb,),
        in_specs=[
            pl.BlockSpec((_BS, D), lambda i: (i, 0)),
            pl.BlockSpec((_BS, _WPAD), lambda i: (i, 0)),
        ],
        out_specs=pl.BlockSpec((_BS, D + _WDIM), lambda i: (i, 0)),
        out_shape=jax.ShapeDtypeStruct((rows, D + _WDIM), jnp.float32),
    )(emb, wemb)
    return out.reshape(B, S, D + _WDIM)
